# Initial kernel scaffold; baseline (speedup 1.0000x reference)
import jax
import jax.numpy as jnp
from jax.experimental import pallas as pl


def kernel(i, j, table):
    raise NotImplementedError("write your pallas kernel here")


# speedup vs baseline: 1.2366x; 1.2366x over previous
import functools
import jax
import jax.numpy as jnp
import numpy as np
from jax import lax
from jax.experimental import pallas as pl
from jax.experimental.pallas import tpu as pltpu
from jax.experimental.pallas import tpu_sc as plsc

NC = 2
NS = 16
NW = NC * NS
L = 16

ROWS = 1024
COLS = 1024
INV = 10

QROW = 128
CH_R = 16


def _compute_base(iv, jv):
  x = iv.astype(jnp.float32) / np.float32(INV)
  y = jv.astype(jnp.float32) / np.float32(INV)
  r = x.astype(jnp.int32)
  c = y.astype(jnp.int32)
  base = r * COLS + c
  return x, y, r, c, base


def _weights(x, r):
  rf = r.astype(jnp.float32)
  df = x - rf
  degen = jnp.logical_and(x == rf, r >= 32)
  dc = jnp.where(degen, np.float32(0.0), (rf + np.float32(1.0)) - x)
  return df, dc


def _body(i_hbm, j_hbm, table_hbm, out_hbm,
          i_v, j_v, idx00, idx01, idx10, idx11,
          g00, g01, g10, g11, out_v, sem, rows_per_w=0, n_chunks=0):
  wid = lax.axis_index("s") * NC + lax.axis_index("c")
  row0 = wid * rows_per_w

  def chunk_body(t, _):
    rbase = row0 + t * CH_R
    pltpu.sync_copy(i_hbm.at[pl.ds(rbase, CH_R)], i_v)
    pltpu.sync_copy(j_hbm.at[pl.ds(rbase, CH_R)], j_v)

    def idx_body(k, _):
      s = k // (QROW // L)
      cc = (k % (QROW // L)) * L
      iv = i_v[s, pl.ds(cc, L)]
      jv = j_v[s, pl.ds(cc, L)]
      _, _, _, _, base = _compute_base(iv, jv)
      idx00[s, pl.ds(cc, L)] = base
      idx01[s, pl.ds(cc, L)] = base + 1
      idx10[s, pl.ds(cc, L)] = base + COLS
      idx11[s, pl.ds(cc, L)] = base + (COLS + 1)
      return 0

    lax.fori_loop(0, CH_R * (QROW // L), idx_body, 0)

    copies = []
    for s in range(CH_R):
      copies.append(pltpu.async_copy(table_hbm.at[idx00.at[s]], g00.at[s], sem))
      copies.append(pltpu.async_copy(table_hbm.at[idx01.at[s]], g01.at[s], sem))
      copies.append(pltpu.async_copy(table_hbm.at[idx10.at[s]], g10.at[s], sem))
      copies.append(pltpu.async_copy(table_hbm.at[idx11.at[s]], g11.at[s], sem))
    for cp in copies:
      cp.wait()

    def comb_body(k, _):
      s = k // (QROW // L)
      cc = (k % (QROW // L)) * L
      iv = i_v[s, pl.ds(cc, L)]
      jv = j_v[s, pl.ds(cc, L)]
      x, y, r, c, _ = _compute_base(iv, jv)
      i_df, i_dc = _weights(x, r)
      j_df, j_dc = _weights(y, c)
      o = g00[s, pl.ds(cc, L)]
      tt = g01[s, pl.ds(cc, L)]
      rr = g10[s, pl.ds(cc, L)]
      rt = g11[s, pl.ds(cc, L)]
      out_v[s, pl.ds(cc, L)] = (j_dc * (i_dc * o + i_df * rr)
                                + j_df * (i_dc * tt + i_df * rt))
      return 0

    lax.fori_loop(0, CH_R * (QROW // L), comb_body, 0)

    pltpu.sync_copy(out_v, out_hbm.at[pl.ds(rbase, CH_R)])
    return 0

  lax.fori_loop(0, n_chunks, chunk_body, 0)


@jax.jit
def kernel(i, j, table):
  n = i.shape[0] * i.shape[1]
  assert n % (NW * CH_R * QROW) == 0
  n_rows = n // QROW
  rows_per_w = n_rows // NW
  n_chunks = rows_per_w // CH_R

  i2 = i.reshape(n_rows, QROW)
  j2 = j.reshape(n_rows, QROW)
  tflat = table.reshape(-1)

  mesh = plsc.VectorSubcoreMesh(core_axis_name="c", subcore_axis_name="s",
                                num_cores=NC, num_subcores=NS)
  body = functools.partial(_body, rows_per_w=rows_per_w, n_chunks=n_chunks)
  out = pl.kernel(
      body,
      out_type=jax.ShapeDtypeStruct((n_rows, QROW), jnp.float32),
      mesh=mesh,
      scratch_types=[
          pltpu.VMEM((CH_R, QROW), jnp.int32),
          pltpu.VMEM((CH_R, QROW), jnp.int32),
          pltpu.VMEM((CH_R, QROW), jnp.int32),
          pltpu.VMEM((CH_R, QROW), jnp.int32),
          pltpu.VMEM((CH_R, QROW), jnp.int32),
          pltpu.VMEM((CH_R, QROW), jnp.int32),
          pltpu.VMEM((CH_R, QROW), jnp.float32),
          pltpu.VMEM((CH_R, QROW), jnp.float32),
          pltpu.VMEM((CH_R, QROW), jnp.float32),
          pltpu.VMEM((CH_R, QROW), jnp.float32),
          pltpu.VMEM((CH_R, QROW), jnp.float32),
          pltpu.SemaphoreType.DMA,
      ],
  )(i2, j2, tflat)
  return out.reshape(i.shape)


# speedup vs baseline: 1.2381x; 1.0013x over previous
import functools
import jax
import jax.numpy as jnp
import numpy as np
from jax import lax
from jax.experimental import pallas as pl
from jax.experimental.pallas import tpu as pltpu
from jax.experimental.pallas import tpu_sc as plsc

NC = 2
NS = 16
NW = NC * NS
L = 16

ROWS = 1024
COLS = 1024
INV = 10

QROW = 128
CH_R = 16


def _compute_base(iv, jv):
  x = iv.astype(jnp.float32) / np.float32(INV)
  y = jv.astype(jnp.float32) / np.float32(INV)
  r = x.astype(jnp.int32)
  c = y.astype(jnp.int32)
  base = r * COLS + c
  return x, y, r, c, base


def _weights(x, r):
  rf = r.astype(jnp.float32)
  df = x - rf
  degen = jnp.logical_and(x == rf, r >= 32)
  dc = jnp.where(degen, np.float32(0.0), (rf + np.float32(1.0)) - x)
  return df, dc


def _body(i_hbm, j_hbm, table_hbm, out_hbm,
          i_v, j_v, idx00, idx01, idx10, idx11,
          g00, g01, g10, g11, out_v, sem, rows_per_w=0, n_chunks=0):
  wid = lax.axis_index("s") * NC + lax.axis_index("c")
  row0 = wid * rows_per_w

  def chunk_body(t, _):
    rbase = row0 + t * CH_R
    pltpu.sync_copy(i_hbm.at[pl.ds(rbase, CH_R)], i_v)
    pltpu.sync_copy(j_hbm.at[pl.ds(rbase, CH_R)], j_v)

    def idx_body(k, _):
      s = k // (QROW // L)
      cc = (k % (QROW // L)) * L
      iv = i_v[s, pl.ds(cc, L)]
      jv = j_v[s, pl.ds(cc, L)]
      _, _, _, _, base = _compute_base(iv, jv)
      f = k * L
      idx00[pl.ds(f, L)] = base
      idx01[pl.ds(f, L)] = base + 1
      idx10[pl.ds(f, L)] = base + COLS
      idx11[pl.ds(f, L)] = base + (COLS + 1)
      return 0

    lax.fori_loop(0, CH_R * (QROW // L), idx_body, 0)

    copies = [
        pltpu.async_copy(table_hbm.at[idx00], g00, sem),
        pltpu.async_copy(table_hbm.at[idx01], g01, sem),
        pltpu.async_copy(table_hbm.at[idx10], g10, sem),
        pltpu.async_copy(table_hbm.at[idx11], g11, sem),
    ]
    for cp in copies:
      cp.wait()

    def comb_body(k, _):
      s = k // (QROW // L)
      cc = (k % (QROW // L)) * L
      iv = i_v[s, pl.ds(cc, L)]
      jv = j_v[s, pl.ds(cc, L)]
      x, y, r, c, _ = _compute_base(iv, jv)
      i_df, i_dc = _weights(x, r)
      j_df, j_dc = _weights(y, c)
      f = k * L
      o = g00[pl.ds(f, L)]
      tt = g01[pl.ds(f, L)]
      rr = g10[pl.ds(f, L)]
      rt = g11[pl.ds(f, L)]
      out_v[s, pl.ds(cc, L)] = (j_dc * (i_dc * o + i_df * rr)
                                + j_df * (i_dc * tt + i_df * rt))
      return 0

    lax.fori_loop(0, CH_R * (QROW // L), comb_body, 0)

    pltpu.sync_copy(out_v, out_hbm.at[pl.ds(rbase, CH_R)])
    return 0

  lax.fori_loop(0, n_chunks, chunk_body, 0)


@jax.jit
def kernel(i, j, table):
  n = i.shape[0] * i.shape[1]
  assert n % (NW * CH_R * QROW) == 0
  n_rows = n // QROW
  rows_per_w = n_rows // NW
  n_chunks = rows_per_w // CH_R

  i2 = i.reshape(n_rows, QROW)
  j2 = j.reshape(n_rows, QROW)
  tflat = table.reshape(-1)

  mesh = plsc.VectorSubcoreMesh(core_axis_name="c", subcore_axis_name="s",
                                num_cores=NC, num_subcores=NS)
  body = functools.partial(_body, rows_per_w=rows_per_w, n_chunks=n_chunks)
  out = pl.kernel(
      body,
      out_type=jax.ShapeDtypeStruct((n_rows, QROW), jnp.float32),
      mesh=mesh,
      scratch_types=[
          pltpu.VMEM((CH_R, QROW), jnp.int32),
          pltpu.VMEM((CH_R, QROW), jnp.int32),
          pltpu.VMEM((CH_R * QROW,), jnp.int32),
          pltpu.VMEM((CH_R * QROW,), jnp.int32),
          pltpu.VMEM((CH_R * QROW,), jnp.int32),
          pltpu.VMEM((CH_R * QROW,), jnp.int32),
          pltpu.VMEM((CH_R * QROW,), jnp.float32),
          pltpu.VMEM((CH_R * QROW,), jnp.float32),
          pltpu.VMEM((CH_R * QROW,), jnp.float32),
          pltpu.VMEM((CH_R * QROW,), jnp.float32),
          pltpu.VMEM((CH_R, QROW), jnp.float32),
          pltpu.SemaphoreType.DMA,
      ],
  )(i2, j2, tflat)
  return out.reshape(i.shape)


# speedup vs baseline: 1.4747x; 1.1910x over previous
import jax
import jax.numpy as jnp
import numpy as np
from jax import lax
from jax.experimental import pallas as pl
from jax.experimental.pallas import tpu as pltpu
from jax.experimental.pallas import tpu_sc as plsc

NC = 2
NS = 16
NW = NC * NS
L = 16

COLS = 1024
QCH = 3200
NV = QCH // L
UNROLL = 8


def _index_math(iv, jv):
  r = (iv * 6554) >> 16
  c = (jv * 6554) >> 16
  di = iv - r * 10
  dj = jv - c * 10
  base = r * COLS + c
  i_df = di.astype(jnp.float32) * np.float32(0.1)
  j_df = dj.astype(jnp.float32) * np.float32(0.1)
  one = np.float32(1.0)
  zero = np.float32(0.0)
  i_dc = jnp.where(jnp.logical_and(di == 0, r >= 32), zero, one - i_df)
  j_dc = jnp.where(jnp.logical_and(dj == 0, c >= 32), zero, one - j_df)
  return base, i_df, i_dc, j_df, j_dc


def _body(i_hbm, j_hbm, table_hbm, out_hbm, *refs):
  nsets = 2
  per = 17
  sets = []
  for s in range(nsets):
    (i_v, j_v, idx00, idx01, idx10, idx11, g00, g01, g10, g11,
     widf, widc, wjdf, wjdc, out_v, sem_in, sem_g) = refs[s * per:(s + 1) * per]
    sets.append(dict(i_v=i_v, j_v=j_v,
                     idx=(idx00, idx01, idx10, idx11),
                     g=(g00, g01, g10, g11),
                     w=(widf, widc, wjdf, wjdc),
                     out_v=out_v, sem_in=sem_in, sem_g=sem_g))
  n_chunks = refs[nsets * per]

  wid = lax.axis_index("s") * NC + lax.axis_index("c")
  q0 = wid * (n_chunks * QCH)

  def front(t, S):
    qb = q0 + t * QCH
    ci = pltpu.async_copy(i_hbm.at[pl.ds(qb, QCH)], S["i_v"], S["sem_in"])
    cj = pltpu.async_copy(j_hbm.at[pl.ds(qb, QCH)], S["j_v"], S["sem_in"])
    ci.wait()
    cj.wait()

    def comp(k, _):
      for u in range(UNROLL):
        f = (k * UNROLL + u) * L
        sl = pl.ds(f, L)
        iv = S["i_v"][sl]
        jv = S["j_v"][sl]
        base, i_df, i_dc, j_df, j_dc = _index_math(iv, jv)
        S["idx"][0][sl] = base
        S["idx"][1][sl] = base + 1
        S["idx"][2][sl] = base + COLS
        S["idx"][3][sl] = base + (COLS + 1)
        S["w"][0][sl] = i_df
        S["w"][1][sl] = i_dc
        S["w"][2][sl] = j_df
        S["w"][3][sl] = j_dc
      return 0

    lax.fori_loop(0, NV // UNROLL, comp, 0)
    for a in range(4):
      pltpu.async_copy(table_hbm.at[S["idx"][a]], S["g"][a], S["sem_g"])

  def back(t, S):
    qb = q0 + t * QCH
    for a in range(4):
      pltpu.make_async_copy(table_hbm.at[S["idx"][a]], S["g"][a],
                            S["sem_g"]).wait()

    def comb(k, _):
      for u in range(UNROLL):
        f = (k * UNROLL + u) * L
        sl = pl.ds(f, L)
        o = S["g"][0][sl]
        tt = S["g"][1][sl]
        rr = S["g"][2][sl]
        rt = S["g"][3][sl]
        i_df = S["w"][0][sl]
        i_dc = S["w"][1][sl]
        j_df = S["w"][2][sl]
        j_dc = S["w"][3][sl]
        S["out_v"][sl] = (j_dc * (i_dc * o + i_df * rr)
                          + j_df * (i_dc * tt + i_df * rt))
      return 0

    lax.fori_loop(0, NV // UNROLL, comb, 0)
    pltpu.sync_copy(S["out_v"], out_hbm.at[pl.ds(qb, QCH)])

  A, B = sets

  front(0, A)

  def pair(u, _):
    t0 = 2 * u
    front(t0 + 1, B)
    back(t0, A)
    front(t0 + 2, A)
    back(t0 + 1, B)
    return 0

  lax.fori_loop(0, n_chunks // 2 - 1, pair, 0)
  tl = n_chunks - 2
  front(tl + 1, B)
  back(tl, A)
  back(tl + 1, B)


@jax.jit
def kernel(i, j, table):
  n = i.shape[0] * i.shape[1]
  assert n % (NW * QCH) == 0
  n_chunks = n // (NW * QCH)
  assert n_chunks % 2 == 0

  i1 = i.reshape(-1)
  j1 = j.reshape(-1)
  tflat = table.reshape(-1)

  mesh = plsc.VectorSubcoreMesh(core_axis_name="c", subcore_axis_name="s",
                                num_cores=NC, num_subcores=NS)

  def set_types():
    return ([pltpu.VMEM((QCH,), jnp.int32)] * 6
            + [pltpu.VMEM((QCH,), jnp.float32)] * 9
            + [pltpu.SemaphoreType.DMA] * 2)

  body = lambda *a: _body(*a, n_chunks)
  out = pl.kernel(
      body,
      out_type=jax.ShapeDtypeStruct((n,), jnp.float32),
      mesh=mesh,
      scratch_types=set_types() + set_types(),
  )(i1, j1, tflat)
  return out.reshape(i.shape)
